# baseline (speedup 1.0000x reference)
import functools

import jax
import jax.numpy as jnp
from jax import lax
from jax.experimental import pallas as pl
from jax.experimental.pallas import tpu as pltpu
from jax.experimental.pallas import tpu_sc as plsc


def _gather_body(nc, b_per_w, dim, table2_hbm, idx_hbm, out_hbm,
                 idx_v, idx2_v, rows2_v, row_buf, sem):
    wid = lax.axis_index("s") * nc + lax.axis_index("c")
    base = pl.multiple_of(wid * b_per_w, b_per_w)
    two_d = 2 * dim
    pltpu.sync_copy(idx_hbm.at[pl.ds(base, b_per_w)], idx_v)
    nv = b_per_w // 16
    for r in range(nv):
        idx2_v[pl.ds(16 * r, 16)] = idx_v[pl.ds(16 * r, 16)] >> 1
    pltpu.async_copy(table2_hbm.at[idx2_v], rows2_v, sem).wait()

    def extract(j, carry):
        for k in range(nv):
            v = idx_v[pl.ds(16 * k, 16)]
            half_off = (v & 1) * dim
            rid = lax.iota(jnp.int32, 16) + 16 * k
            vals = plsc.load_gather(rows2_v, [rid, half_off + j])
            flat = rid * dim + j
            plsc.store_scatter(row_buf, [flat >> 7, flat & 127], vals)
        return carry

    lax.fori_loop(0, dim, extract, 0, unroll=False)
    n_pair_rows = b_per_w * dim // two_d
    pair_base = pl.multiple_of(wid * n_pair_rows, n_pair_rows)
    pltpu.sync_copy(row_buf, out_hbm.at[pl.ds(pair_base, n_pair_rows)])


def _sc_gather(embeddings, idx):
    vocab, dim = embeddings.shape
    batch = idx.shape[0]
    info = plsc.get_sparse_core_info()
    nc, ns = info.num_cores, info.num_subcores
    nw = nc * ns
    b_per_w = batch // nw
    table2 = embeddings.reshape(vocab // 2, 2 * dim)
    mesh = plsc.VectorSubcoreMesh(core_axis_name="c", subcore_axis_name="s")
    k = pl.kernel(
        functools.partial(_gather_body, nc, b_per_w, dim),
        out_type=jax.ShapeDtypeStruct((batch * dim // (2 * dim), 2 * dim),
                                      jnp.float32),
        mesh=mesh,
        scratch_types=[
            pltpu.VMEM((b_per_w,), jnp.int32),
            pltpu.VMEM((b_per_w,), jnp.int32),
            pltpu.VMEM((b_per_w, 2 * dim), jnp.float32),
            pltpu.VMEM((b_per_w * dim // (2 * dim), 2 * dim), jnp.float32),
            pltpu.SemaphoreType.DMA,
        ],
        compiler_params=pltpu.CompilerParams(
            use_tc_tiling_on_sc=True, needs_layout_passes=False),
    )
    out_pairs = k(table2, idx)
    return out_pairs.reshape(batch, dim)


def _proj_body(wt_ref, e_ref, b_ref, o_ref):
    o_ref[...] = lax.dot_general(
        wt_ref[...], e_ref[...], (((0,), (1,)), ((), ())),
        preferred_element_type=jnp.float32,
    ) + jnp.transpose(b_ref[...])


def _tc_project_t(embed, Wt, b_row, vt=1024):
    batch, dim = embed.shape
    vocab = Wt.shape[1]
    nvt = pl.cdiv(vocab, vt)
    return pl.pallas_call(
        _proj_body,
        grid=(nvt,),
        in_specs=[
            pl.BlockSpec((dim, vt), lambda j: (0, j)),
            pl.BlockSpec((batch, dim), lambda j: (0, 0)),
            pl.BlockSpec((1, vt), lambda j: (0, j)),
        ],
        out_specs=pl.BlockSpec((vt, batch), lambda j: (j, 0)),
        out_shape=jax.ShapeDtypeStruct((vocab, batch), jnp.float32),
        compiler_params=pltpu.CompilerParams(
            dimension_semantics=("arbitrary",),
            vmem_limit_bytes=48 * 1024 * 1024,
        ),
    )(Wt, embed, b_row)


def kernel(target_word_idx, embeddings, W, b):
    idx = target_word_idx.astype(jnp.int32)
    embed = _sc_gather(embeddings, idx)
    out_t = _tc_project_t(embed, W.T, b.reshape(1, -1))
    return out_t.T
